# baseline (speedup 1.0000x reference)
import jax
import jax.numpy as jnp
from jax import lax
from jax.experimental import pallas as pl
from jax.experimental.pallas import tpu as pltpu

B, C, H, W = 64, 384, 32, 32
HW = H * W
E = 8
RED = 48
BB = 8


def _body(x_ref, w1_hbm, b1_hbm, w2_hbm, b2_hbm, out_ref, wn_ref,
          w1_ref, b1_ref, w2_ref, b2_ref, wsem):
    i = pl.program_id(0)

    @pl.when(i == 0)
    def _():
        for k, (src, dst) in enumerate([(w1_hbm, w1_ref), (b1_hbm, b1_ref),
                                        (w2_hbm, w2_ref), (b2_hbm, b2_ref)]):
            pltpu.make_async_copy(src, dst, wsem.at[k]).start()
        for k, (src, dst) in enumerate([(w1_hbm, w1_ref), (b1_hbm, b1_ref),
                                        (w2_hbm, w2_ref), (b2_hbm, b2_ref)]):
            pltpu.make_async_copy(src, dst, wsem.at[k]).wait()

    xs = x_ref[...]
    n = H
    while n > 1:
        n //= 2
        xs = xs[:, :n] + xs[:, n:2 * n]
    ys = xs[:, 0]
    n = W
    while n > 1:
        n //= 2
        ys = ys[:, :n] + ys[:, n:2 * n]
    pooled = ys[:, 0] * (1.0 / HW)
    h = jnp.dot(pooled, w1_ref[...], preferred_element_type=jnp.float32)
    h = h + b1_ref[...]
    h = h * jax.nn.sigmoid(h)
    logits = jnp.dot(h, w2_ref[...], preferred_element_type=jnp.float32)
    logits = logits + b2_ref[...]
    w = jax.nn.softmax(logits, axis=1)
    idx = lax.broadcasted_iota(jnp.int32, (BB, E), 1)
    m1 = jnp.max(w, axis=1, keepdims=True)
    i1 = jnp.min(jnp.where(w == m1, idx, E), axis=1, keepdims=True)
    w_rest = jnp.where(idx == i1, -jnp.inf, w)
    m2 = jnp.max(w_rest, axis=1, keepdims=True)
    i2 = jnp.min(jnp.where(w_rest == m2, idx, E), axis=1, keepdims=True)
    mask = (idx == i1) | (idx == i2)
    wsel = jnp.where(mask, w, 0.0)
    wn = wsel / (jnp.sum(wsel, axis=1, keepdims=True) + 1e-8)
    wn_ref[pl.ds(i * BB, BB), :] = wn

    @pl.when(i == B // BB - 1)
    def _():
        wnt = wn_ref[...].T
        out_ref[...] = jnp.broadcast_to(wnt[:, None, None, :], (E, H, W, B))


@jax.jit
def kernel(x, W1, b1, W2, b2):
    xt = jnp.transpose(x, (0, 2, 3, 1))
    pout = pl.pallas_call(
        _body,
        grid=(B // BB,),
        in_specs=[
            pl.BlockSpec((BB, H, W, C), lambda i: (i, 0, 0, 0)),
            pl.BlockSpec(memory_space=pltpu.HBM),
            pl.BlockSpec(memory_space=pltpu.HBM),
            pl.BlockSpec(memory_space=pltpu.HBM),
            pl.BlockSpec(memory_space=pltpu.HBM),
        ],
        out_specs=pl.BlockSpec((E, H, W, B), lambda i: (0, 0, 0, 0)),
        out_shape=jax.ShapeDtypeStruct((E, H, W, B), jnp.float32),
        scratch_shapes=[
            pltpu.VMEM((B, E), jnp.float32),
            pltpu.VMEM((C, RED), jnp.float32),
            pltpu.VMEM((1, RED), jnp.float32),
            pltpu.VMEM((RED, E), jnp.float32),
            pltpu.VMEM((1, E), jnp.float32),
            pltpu.SemaphoreType.DMA((4,)),
        ],
    )(xt, W1, b1.reshape(1, RED), W2, b2.reshape(1, E))
    return jnp.transpose(pout, (3, 0, 1, 2))


# speedup vs baseline: 1.0065x; 1.0065x over previous
import jax
import jax.numpy as jnp
from jax import lax
from jax.experimental import pallas as pl
from jax.experimental.pallas import tpu as pltpu

B, C, H, W = 64, 384, 32, 32
HW = H * W
E = 8
RED = 48
BB = 4


def _body(x_ref, w1_ref, b1_ref, w2_ref, b2_ref, out_ref, wn_ref):
    i = pl.program_id(0)
    xs = x_ref[...]
    n = H
    while n > 1:
        n //= 2
        xs = xs[:, :n] + xs[:, n:2 * n]
    ys = xs[:, 0]
    n = W
    while n > 1:
        n //= 2
        ys = ys[:, :n] + ys[:, n:2 * n]
    pooled = ys[:, 0] * (1.0 / HW)
    h = jnp.dot(pooled, w1_ref[...], preferred_element_type=jnp.float32)
    h = h + b1_ref[...]
    h = h * jax.nn.sigmoid(h)
    logits = jnp.dot(h, w2_ref[...], preferred_element_type=jnp.float32)
    logits = logits + b2_ref[...]
    w = jax.nn.softmax(logits, axis=1)
    idx = lax.broadcasted_iota(jnp.int32, (BB, E), 1)
    m1 = jnp.max(w, axis=1, keepdims=True)
    i1 = jnp.min(jnp.where(w == m1, idx, E), axis=1, keepdims=True)
    w_rest = jnp.where(idx == i1, -jnp.inf, w)
    m2 = jnp.max(w_rest, axis=1, keepdims=True)
    i2 = jnp.min(jnp.where(w_rest == m2, idx, E), axis=1, keepdims=True)
    mask = (idx == i1) | (idx == i2)
    wsel = jnp.where(mask, w, 0.0)
    wn = wsel / (jnp.sum(wsel, axis=1, keepdims=True) + 1e-8)
    wn_ref[pl.ds(i * BB, BB), :] = wn

    @pl.when(i == B // BB - 1)
    def _():
        wnt = wn_ref[...].T
        out_ref[...] = jnp.broadcast_to(wnt[:, None, None, :], (E, H, W, B))


@jax.jit
def kernel(x, W1, b1, W2, b2):
    xt = jnp.transpose(x, (0, 2, 3, 1))
    pout = pl.pallas_call(
        _body,
        grid=(B // BB,),
        in_specs=[
            pl.BlockSpec((BB, H, W, C), lambda i: (i, 0, 0, 0)),
            pl.BlockSpec((C, RED), lambda i: (0, 0)),
            pl.BlockSpec((1, RED), lambda i: (0, 0)),
            pl.BlockSpec((RED, E), lambda i: (0, 0)),
            pl.BlockSpec((1, E), lambda i: (0, 0)),
        ],
        out_specs=pl.BlockSpec((E, H, W, B), lambda i: (0, 0, 0, 0)),
        out_shape=jax.ShapeDtypeStruct((E, H, W, B), jnp.float32),
        scratch_shapes=[pltpu.VMEM((B, E), jnp.float32)],
    )(xt, W1, b1.reshape(1, RED), W2, b2.reshape(1, E))
    return jnp.transpose(pout, (3, 0, 1, 2))
